# baseline (speedup 1.0000x reference)
import functools

import jax
import jax.numpy as jnp
from jax import lax
from jax.experimental import pallas as pl
from jax.experimental.pallas import tpu as pltpu
from jax.experimental.pallas import tpu_sc as plsc

EPS = 1e-5
NC = 2
NS = 16
NW = NC * NS
SUB = 32
NBUF = 3
CHUNK_SIZES = (512, 768, 1024, 1024, 512, 256)
BLK = 256


def _sc_gather(table, ids_flat, batch, seq, seq0, seq_c, hidden):
    n = batch * seq_c
    rows_per_tile = n // NW
    tiles_per_b = NW // batch
    nsub = rows_per_tile // SUB
    mesh = plsc.VectorSubcoreMesh(core_axis_name="c", subcore_axis_name="s")

    @functools.partial(
        pl.kernel,
        mesh=mesh,
        out_type=jax.ShapeDtypeStruct((n, hidden), jnp.float32),
        scratch_types=[
            pltpu.VMEM((rows_per_tile,), jnp.int32),
        ] + [pltpu.VMEM((SUB, hidden), jnp.float32) for _ in range(NBUF)]
          + [pltpu.SemaphoreType.DMA for _ in range(2 * NBUF)],
    )
    def k(table_hbm, ids_hbm, out_hbm, idx_v, *rest):
        bufs = rest[:NBUF]
        gsems = rest[NBUF:2 * NBUF]
        ssems = rest[2 * NBUF:]
        wid = lax.axis_index("s") * NC + lax.axis_index("c")
        b = wid // tiles_per_b
        flat0 = b * seq + seq0 + (wid % tiles_per_b) * rows_per_tile
        base = wid * rows_per_tile
        pltpu.sync_copy(ids_hbm.at[pl.ds(flat0, rows_per_tile)], idx_v)

        pend = [None] * nsub
        for c in range(min(NBUF, nsub)):
            pend[c] = pltpu.async_copy(
                table_hbm.at[idx_v.at[pl.ds(c * SUB, SUB)]], bufs[c % NBUF],
                gsems[c % NBUF])
        for c in range(nsub):
            r = c % NBUF
            pend[c].wait()
            pltpu.async_copy(
                bufs[r], out_hbm.at[pl.ds(base + c * SUB, SUB)], ssems[r]
            ).wait()
            if c + NBUF < nsub:
                pend[c + NBUF] = pltpu.async_copy(
                    table_hbm.at[idx_v.at[pl.ds((c + NBUF) * SUB, SUB)]],
                    bufs[r], gsems[r])

    return k(table, ids_flat)


def _ln_body(prev_ref, g_ref, p_ref, w_ref, b_ref, o_ref):
    del prev_ref
    x = g_ref[...] + p_ref[...]
    m = jnp.mean(x, axis=-1, keepdims=True)
    xc = x - m
    v = jnp.mean(xc * xc, axis=-1, keepdims=True)
    o_ref[...] = xc * lax.rsqrt(v + EPS) * w_ref[...] + b_ref[...]


def kernel(input_ids, embed_tokens, embed_positions, ln_weight, ln_bias):
    batch, seq = input_ids.shape
    vocab, hidden = embed_tokens.shape
    n = batch * seq
    assert sum(CHUNK_SIZES) == seq

    ids32 = input_ids.astype(jnp.int32)
    w2 = ln_weight.reshape(1, hidden)
    b2 = ln_bias.reshape(1, hidden)

    offsets = []
    off = 0
    for sc_ in CHUNK_SIZES:
        offsets.append(off)
        off += sc_

    ids_flat = ids32.reshape(-1)
    gathered = [
        _sc_gather(embed_tokens, ids_flat, batch, seq, offsets[k],
                   CHUNK_SIZES[k], hidden)
        for k in range(len(CHUNK_SIZES))
    ]

    out = None
    for k in range(len(CHUNK_SIZES)):
        seq_c = CHUNK_SIZES[k]
        seq0 = offsets[k]
        pos_blocks = seq_c // BLK

        def g_map(p, b, _pb=pos_blocks):
            return (b * _pb + p, 0)

        def p_map(p, b, _s0=seq0 // BLK):
            return (_s0 + p, 0)

        def o_map(p, b, _s0=seq0 // BLK):
            return (b * (seq // BLK) + _s0 + p, 0)

        in_specs = [
            pl.BlockSpec(memory_space=pl.ANY),
            pl.BlockSpec((BLK, hidden), g_map),
            pl.BlockSpec((BLK, hidden), p_map),
            pl.BlockSpec((1, hidden), lambda p, b: (0, 0)),
            pl.BlockSpec((1, hidden), lambda p, b: (0, 0)),
        ]
        if out is None:
            prev = jnp.zeros((8, 128), dtype=jnp.float32)
        else:
            prev = out
        out = pl.pallas_call(
            _ln_body,
            grid=(pos_blocks, batch),
            in_specs=in_specs,
            out_specs=pl.BlockSpec((BLK, hidden), o_map),
            out_shape=jax.ShapeDtypeStruct((n, hidden), jnp.float32),
            input_output_aliases={} if out is None else {0: 0},
        )(prev, gathered[k], embed_positions, w2, b2)
    return out.reshape(batch, seq, hidden)


# speedup vs baseline: 1.0569x; 1.0569x over previous
import functools

import jax
import jax.numpy as jnp
from jax import lax
from jax.experimental import pallas as pl
from jax.experimental.pallas import tpu as pltpu
from jax.experimental.pallas import tpu_sc as plsc

EPS = 1e-5
NC = 2
NS = 16
NW = NC * NS
SUB = 32
NBUF = 3
BLK = 512


def _sc_gather(table, ids_flat, hidden):
    n = ids_flat.shape[0]
    rows_per_tile = n // NW
    nsub = rows_per_tile // SUB
    mesh = plsc.VectorSubcoreMesh(core_axis_name="c", subcore_axis_name="s")

    @functools.partial(
        pl.kernel,
        mesh=mesh,
        out_type=jax.ShapeDtypeStruct((n, hidden), jnp.float32),
        scratch_types=[
            pltpu.VMEM((rows_per_tile,), jnp.int32),
        ] + [pltpu.VMEM((SUB, hidden), jnp.float32) for _ in range(NBUF)]
          + [pltpu.SemaphoreType.DMA for _ in range(2 * NBUF)],
    )
    def k(table_hbm, ids_hbm, out_hbm, idx_v, *rest):
        bufs = rest[:NBUF]
        gsems = rest[NBUF:2 * NBUF]
        ssems = rest[2 * NBUF:]
        wid = lax.axis_index("s") * NC + lax.axis_index("c")
        base = wid * rows_per_tile
        pltpu.sync_copy(ids_hbm.at[pl.ds(base, rows_per_tile)], idx_v)

        pend = [None] * nsub
        for c in range(min(NBUF, nsub)):
            pend[c] = pltpu.async_copy(
                table_hbm.at[idx_v.at[pl.ds(c * SUB, SUB)]], bufs[c % NBUF],
                gsems[c % NBUF])
        for c in range(nsub):
            r = c % NBUF
            pend[c].wait()
            pltpu.async_copy(
                bufs[r], out_hbm.at[pl.ds(base + c * SUB, SUB)], ssems[r]
            ).wait()
            if c + NBUF < nsub:
                pend[c + NBUF] = pltpu.async_copy(
                    table_hbm.at[idx_v.at[pl.ds((c + NBUF) * SUB, SUB)]],
                    bufs[r], gsems[r])

    return k(table, ids_flat)


def _ln_body(g_ref, p_ref, w_ref, b_ref, o_ref):
    x = g_ref[...] + p_ref[...]
    m = jnp.mean(x, axis=-1, keepdims=True)
    xc = x - m
    v = jnp.mean(xc * xc, axis=-1, keepdims=True)
    o_ref[...] = xc * lax.rsqrt(v + EPS) * w_ref[...] + b_ref[...]


def kernel(input_ids, embed_tokens, embed_positions, ln_weight, ln_bias):
    batch, seq = input_ids.shape
    vocab, hidden = embed_tokens.shape
    n = batch * seq
    pos_blocks = seq // BLK

    ids_flat = input_ids.astype(jnp.int32).reshape(-1)
    w2 = ln_weight.reshape(1, hidden)
    b2 = ln_bias.reshape(1, hidden)

    gathered = _sc_gather(embed_tokens, ids_flat, hidden)

    out = pl.pallas_call(
        _ln_body,
        grid=(pos_blocks, batch),
        in_specs=[
            pl.BlockSpec((BLK, hidden), lambda p, b: (b * pos_blocks + p, 0)),
            pl.BlockSpec((BLK, hidden), lambda p, b: (p, 0)),
            pl.BlockSpec((1, hidden), lambda p, b: (0, 0)),
            pl.BlockSpec((1, hidden), lambda p, b: (0, 0)),
        ],
        out_specs=pl.BlockSpec(
            (BLK, hidden), lambda p, b: (b * pos_blocks + p, 0)),
        out_shape=jax.ShapeDtypeStruct((n, hidden), jnp.float32),
    )(gathered, embed_positions, w2, b2)
    return out.reshape(batch, seq, hidden)


# speedup vs baseline: 1.1131x; 1.0532x over previous
import functools

import jax
import jax.numpy as jnp
from jax import lax
from jax.experimental import pallas as pl
from jax.experimental.pallas import tpu as pltpu
from jax.experimental.pallas import tpu_sc as plsc

EPS = 1e-5
NC = 2
NS = 16
NW = NC * NS
SUB = 32
NBUF = 3
BLK = 1024


def _sc_gather(table, ids_flat, hidden):
    n = ids_flat.shape[0]
    rows_per_tile = n // NW
    nsub = rows_per_tile // SUB
    mesh = plsc.VectorSubcoreMesh(core_axis_name="c", subcore_axis_name="s")

    @functools.partial(
        pl.kernel,
        mesh=mesh,
        out_type=jax.ShapeDtypeStruct((n, hidden), jnp.float32),
        scratch_types=[
            pltpu.VMEM((rows_per_tile,), jnp.int32),
        ] + [pltpu.VMEM((SUB, hidden), jnp.float32) for _ in range(NBUF)]
          + [pltpu.SemaphoreType.DMA for _ in range(2 * NBUF)],
    )
    def k(table_hbm, ids_hbm, out_hbm, idx_v, *rest):
        bufs = rest[:NBUF]
        gsems = rest[NBUF:2 * NBUF]
        ssems = rest[2 * NBUF:]
        wid = lax.axis_index("s") * NC + lax.axis_index("c")
        base = wid * rows_per_tile
        pltpu.sync_copy(ids_hbm.at[pl.ds(base, rows_per_tile)], idx_v)

        pend = [None] * nsub
        for c in range(min(NBUF, nsub)):
            pend[c] = pltpu.async_copy(
                table_hbm.at[idx_v.at[pl.ds(c * SUB, SUB)]], bufs[c % NBUF],
                gsems[c % NBUF])
        for c in range(nsub):
            r = c % NBUF
            pend[c].wait()
            pltpu.async_copy(
                bufs[r], out_hbm.at[pl.ds(base + c * SUB, SUB)], ssems[r]
            ).wait()
            if c + NBUF < nsub:
                pend[c + NBUF] = pltpu.async_copy(
                    table_hbm.at[idx_v.at[pl.ds((c + NBUF) * SUB, SUB)]],
                    bufs[r], gsems[r])

    return k(table, ids_flat)


def _ln_body(g_ref, p_ref, w_ref, b_ref, o_ref):
    x = g_ref[...] + p_ref[...]
    m = jnp.mean(x, axis=-1, keepdims=True)
    xc = x - m
    v = jnp.mean(xc * xc, axis=-1, keepdims=True)
    o_ref[...] = xc * lax.rsqrt(v + EPS) * w_ref[...] + b_ref[...]


def kernel(input_ids, embed_tokens, embed_positions, ln_weight, ln_bias):
    batch, seq = input_ids.shape
    vocab, hidden = embed_tokens.shape
    n = batch * seq
    pos_blocks = seq // BLK

    ids_flat = input_ids.astype(jnp.int32).reshape(-1)
    w2 = ln_weight.reshape(1, hidden)
    b2 = ln_bias.reshape(1, hidden)

    gathered = _sc_gather(embed_tokens, ids_flat, hidden)

    out = pl.pallas_call(
        _ln_body,
        grid=(pos_blocks, batch),
        in_specs=[
            pl.BlockSpec((BLK, hidden), lambda p, b: (b * pos_blocks + p, 0)),
            pl.BlockSpec((BLK, hidden), lambda p, b: (p, 0)),
            pl.BlockSpec((1, hidden), lambda p, b: (0, 0)),
            pl.BlockSpec((1, hidden), lambda p, b: (0, 0)),
        ],
        out_specs=pl.BlockSpec(
            (BLK, hidden), lambda p, b: (b * pos_blocks + p, 0)),
        out_shape=jax.ShapeDtypeStruct((n, hidden), jnp.float32),
    )(gathered, embed_positions, w2, b2)
    return out.reshape(batch, seq, hidden)


# speedup vs baseline: 1.1417x; 1.0257x over previous
import functools

import jax
import jax.numpy as jnp
from jax import lax
from jax.experimental import pallas as pl
from jax.experimental.pallas import tpu as pltpu
from jax.experimental.pallas import tpu_sc as plsc

EPS = 1e-5
NC = 2
NS = 16
NW = NC * NS
SUB = 32
NBUF = 3
BLK = 2048


def _sc_gather(table, ids_flat, hidden):
    n = ids_flat.shape[0]
    rows_per_tile = n // NW
    nsub = rows_per_tile // SUB
    mesh = plsc.VectorSubcoreMesh(core_axis_name="c", subcore_axis_name="s")

    @functools.partial(
        pl.kernel,
        mesh=mesh,
        out_type=jax.ShapeDtypeStruct((n, hidden), jnp.float32),
        scratch_types=[
            pltpu.VMEM((rows_per_tile,), jnp.int32),
        ] + [pltpu.VMEM((SUB, hidden), jnp.float32) for _ in range(NBUF)]
          + [pltpu.SemaphoreType.DMA for _ in range(2 * NBUF)],
    )
    def k(table_hbm, ids_hbm, out_hbm, idx_v, *rest):
        bufs = rest[:NBUF]
        gsems = rest[NBUF:2 * NBUF]
        ssems = rest[2 * NBUF:]
        wid = lax.axis_index("s") * NC + lax.axis_index("c")
        base = wid * rows_per_tile
        pltpu.sync_copy(ids_hbm.at[pl.ds(base, rows_per_tile)], idx_v)

        pend = [None] * nsub
        for c in range(min(NBUF, nsub)):
            pend[c] = pltpu.async_copy(
                table_hbm.at[idx_v.at[pl.ds(c * SUB, SUB)]], bufs[c % NBUF],
                gsems[c % NBUF])
        for c in range(nsub):
            r = c % NBUF
            pend[c].wait()
            pltpu.async_copy(
                bufs[r], out_hbm.at[pl.ds(base + c * SUB, SUB)], ssems[r]
            ).wait()
            if c + NBUF < nsub:
                pend[c + NBUF] = pltpu.async_copy(
                    table_hbm.at[idx_v.at[pl.ds((c + NBUF) * SUB, SUB)]],
                    bufs[r], gsems[r])

    return k(table, ids_flat)


def _ln_body(g_ref, p_ref, w_ref, b_ref, o_ref):
    x = g_ref[...] + p_ref[...]
    m = jnp.mean(x, axis=-1, keepdims=True)
    xc = x - m
    v = jnp.mean(xc * xc, axis=-1, keepdims=True)
    o_ref[...] = xc * lax.rsqrt(v + EPS) * w_ref[...] + b_ref[...]


def kernel(input_ids, embed_tokens, embed_positions, ln_weight, ln_bias):
    batch, seq = input_ids.shape
    vocab, hidden = embed_tokens.shape
    n = batch * seq
    pos_blocks = seq // BLK

    ids_flat = input_ids.astype(jnp.int32).reshape(-1)
    w2 = ln_weight.reshape(1, hidden)
    b2 = ln_bias.reshape(1, hidden)

    gathered = _sc_gather(embed_tokens, ids_flat, hidden)

    out = pl.pallas_call(
        _ln_body,
        grid=(pos_blocks, batch),
        in_specs=[
            pl.BlockSpec((BLK, hidden), lambda p, b: (b * pos_blocks + p, 0)),
            pl.BlockSpec((BLK, hidden), lambda p, b: (p, 0)),
            pl.BlockSpec((1, hidden), lambda p, b: (0, 0)),
            pl.BlockSpec((1, hidden), lambda p, b: (0, 0)),
        ],
        out_specs=pl.BlockSpec(
            (BLK, hidden), lambda p, b: (b * pos_blocks + p, 0)),
        out_shape=jax.ShapeDtypeStruct((n, hidden), jnp.float32),
    )(gathered, embed_positions, w2, b2)
    return out.reshape(batch, seq, hidden)
